# baseline (speedup 1.0000x reference)
import functools

import jax
import jax.numpy as jnp
from jax.experimental import pallas as pl
from jax.experimental.pallas import tpu as pltpu
from jax.experimental.pallas import tpu_sc as plsc

B, T, D_IN, D_HID, D_OUT = 8, 1024, 512, 512, 256
PITCH_DIM = 8
ART_Q, ART_K, ART_D = 4, 1024, 248
PIT_Q, PIT_K, PIT_D = 2, 256, 8
N_TOK = B * T
BT = 512
GRID = (N_TOK // BT,)
ART_DP = 256


def _dot(a, b):
    return jnp.dot(a, b, preferred_element_type=jnp.float32)


def _rowdot(a, b):
    return jax.lax.dot_general(
        a, b, dimension_numbers=(((1,), (1,)), ((), ())),
        preferred_element_type=jnp.float32)


def _split3(cb):
    mask = jnp.uint32(0xFFFF0000)
    trunc = lambda v: jax.lax.bitcast_convert_type(
        jax.lax.bitcast_convert_type(v, jnp.uint32) & mask, jnp.float32)
    hi = trunc(cb)
    r1 = cb - hi
    mid = trunc(r1)
    lo = r1 - mid
    return hi, mid, lo


def _encoder_kernel(x_ref, W0_ref, b0_ref, Wa1_ref, ba1_ref, Wb1_ref, bb1_ref,
                    W1_ref, b1_ref, Wa2_ref, ba2_ref, Wb2_ref, bb2_ref,
                    Wout_ref, bout_ref, t_ref):
    h = _dot(x_ref[...], W0_ref[...]) + b0_ref[...]
    h = jnp.maximum(_dot(h, Wa1_ref[...]) + ba1_ref[...], 0.0)
    h = _dot(h, Wb1_ref[...]) + bb1_ref[...]
    h = _dot(h, W1_ref[...]) + b1_ref[...]
    h = jnp.maximum(_dot(h, Wa2_ref[...]) + ba2_ref[...], 0.0)
    h = _dot(h, Wb2_ref[...]) + bb2_ref[...]
    t_ref[...] = _dot(h, Wout_ref[...]) + bout_ref[...]


def _argmin_kernel(kdim, r_ref, r2_ref, cb_ref, c2_ref, ind_ref):
    dist = (r2_ref[...] - 2.0 * _rowdot(r_ref[...], cb_ref[...])) + c2_ref[...]
    md = jnp.min(dist, axis=1, keepdims=True)
    kiota = jax.lax.broadcasted_iota(jnp.int32, dist.shape, 1)
    ind_ref[...] = jnp.min(jnp.where(dist == md, kiota, kdim),
                           axis=1, keepdims=True)


def _pitch_stage_kernel(kdim, r_ref, r2_ref, cb_ref, hi_ref, mid_ref, lo_ref,
                        c2_ref, ind_ref, q_ref, rn_ref):
    r = r_ref[...]
    dist = (r2_ref[...] - 2.0 * _rowdot(r, cb_ref[...])) + c2_ref[...]
    md = jnp.min(dist, axis=1, keepdims=True)
    kiota = jax.lax.broadcasted_iota(jnp.int32, dist.shape, 1)
    ind = jnp.min(jnp.where(dist == md, kiota, kdim), axis=1, keepdims=True)
    onehot = (kiota == ind).astype(jnp.float32)
    q = (_dot(onehot, hi_ref[...]) + _dot(onehot, mid_ref[...])) \
        + _dot(onehot, lo_ref[...])
    ind_ref[...] = ind
    q_ref[...] = q
    rn_ref[...] = r - q


def _final_kernel(aq_ref, p1_ref, p2_ref, quant_ref):
    pacc = p1_ref[...] + p2_ref[...]
    q = jnp.concatenate([aq_ref[...], pacc], axis=1)
    qa = q[:, :ART_D]
    qp = q[:, ART_D:]
    na = jnp.sqrt((qa ** 2).sum(-1, keepdims=True) + 1e-5)
    na = jnp.where(na == 0.0, 1.0, na)
    npn = jnp.sqrt((qp ** 2).sum(-1, keepdims=True) + 1e-5)
    npn = jnp.where(npn == 0.0, 1.0, npn)
    quant_ref[...] = jnp.concatenate([qa / na, qp / npn], axis=1)


_CP = pltpu.CompilerParams(dimension_semantics=("arbitrary",))
_row_spec = lambda w: pl.BlockSpec((BT, w), lambda i: (i, 0))
_w_spec = lambda a: pl.BlockSpec(a.shape, lambda i: (0,) * a.ndim)


def _argmin_stage(r, r2, cb, c2, kdim, ddim):
    return pl.pallas_call(
        functools.partial(_argmin_kernel, kdim),
        grid=GRID,
        in_specs=[_row_spec(ddim), _row_spec(1), _w_spec(cb), _w_spec(c2)],
        out_specs=_row_spec(1),
        out_shape=jax.ShapeDtypeStruct((N_TOK, 1), jnp.int32),
        compiler_params=_CP,
    )(r, r2, cb, c2)


def _sc_gather(table, idx):
    info = plsc.get_sparse_core_info()
    ncores, nsub = info.num_cores, info.num_subcores
    nw = ncores * nsub
    bpw = N_TOK // nw
    dim = table.shape[1]
    mesh = plsc.VectorSubcoreMesh(core_axis_name="c", subcore_axis_name="s")

    @functools.partial(
        pl.kernel, mesh=mesh,
        out_type=jax.ShapeDtypeStruct((N_TOK, dim), jnp.float32),
        scratch_types=[
            pltpu.VMEM((bpw,), jnp.int32),
            pltpu.VMEM((bpw, dim), jnp.float32),
            pltpu.SemaphoreType.DMA,
        ],
    )
    def k(table_hbm, idx_hbm, out_hbm, idx_v, rows_v, sem):
        wid = jax.lax.axis_index("s") * ncores + jax.lax.axis_index("c")
        base = wid * bpw
        pltpu.sync_copy(idx_hbm.at[pl.ds(base, bpw)], idx_v)
        pltpu.async_copy(table_hbm.at[idx_v], rows_v, sem).wait()
        pltpu.sync_copy(rows_v, out_hbm.at[pl.ds(base, bpw)])

    return k(table, idx)


def _unit_norm(x):
    norm = jnp.sqrt((x ** 2).sum(-1, keepdims=True) + 1e-05)
    norm = jnp.where(norm == 0, 1.0, norm)
    return x / norm


@jax.jit
def kernel(token, W0, b0, Wa1, ba1, Wb1, bb1, W1, b1, Wa2, ba2, Wb2, bb2,
           Wout, bout, art_codebooks, pitch_codebooks):
    non_blank_mask = (token ** 2).sum(-1) > 0
    x = _unit_norm(token).reshape(N_TOK, D_IN)

    row2 = lambda v: v.reshape(1, -1)
    enc_args = (W0, row2(b0), Wa1, row2(ba1), Wb1, row2(bb1), W1, row2(b1),
                Wa2, row2(ba2), Wb2, row2(bb2), Wout, row2(bout))
    t_pre = pl.pallas_call(
        _encoder_kernel,
        grid=GRID,
        in_specs=[_row_spec(D_IN)] + [_w_spec(a) for a in enc_args],
        out_specs=_row_spec(D_OUT),
        out_shape=jax.ShapeDtypeStruct((N_TOK, D_OUT), jnp.float32),
        compiler_params=_CP,
    )(x, *enc_args)

    t = jnp.concatenate(
        [_unit_norm(t_pre[..., :-PITCH_DIM]), _unit_norm(t_pre[..., -PITCH_DIM:])], -1)
    t = jnp.where(non_blank_mask.reshape(N_TOK)[..., None], t, 0.0)

    inds = []
    loss = jnp.asarray(0.0, jnp.float32)

    acb_pad = jnp.pad(art_codebooks, ((0, 0), (0, 0), (0, ART_DP - ART_D)))
    res = t[:, :ART_D]
    art_q = None
    for i in range(ART_Q):
        cb = art_codebooks[i]
        c2 = (cb ** 2).sum(-1).reshape(1, ART_K)
        r2 = (res ** 2).sum(-1, keepdims=True)
        ind = _argmin_stage(res, r2, cb, c2, ART_K, ART_D)
        q = _sc_gather(acb_pad[i], ind.reshape(N_TOK))[:, :ART_D]
        inds.append(ind)
        art_q = q if art_q is None else art_q + q
        res = res - q
        loss = loss + jnp.mean(res ** 2)

    pit_res = t[:, ART_D:]
    p_qs = []
    for j in range(PIT_Q):
        cb = pitch_codebooks[j]
        hi, mid, lo = _split3(cb)
        c2 = (cb ** 2).sum(-1).reshape(1, PIT_K)
        r2 = (pit_res ** 2).sum(-1, keepdims=True)
        ind, q, pit_res = pl.pallas_call(
            functools.partial(_pitch_stage_kernel, PIT_K),
            grid=GRID,
            in_specs=[_row_spec(PIT_D), _row_spec(1), _w_spec(cb), _w_spec(hi),
                      _w_spec(mid), _w_spec(lo), _w_spec(c2)],
            out_specs=[_row_spec(1), _row_spec(PIT_D), _row_spec(PIT_D)],
            out_shape=[
                jax.ShapeDtypeStruct((N_TOK, 1), jnp.int32),
                jax.ShapeDtypeStruct((N_TOK, PIT_D), jnp.float32),
                jax.ShapeDtypeStruct((N_TOK, PIT_D), jnp.float32),
            ],
            compiler_params=_CP,
        )(pit_res, r2, cb, hi, mid, lo, c2)
        inds.append(ind)
        p_qs.append(q)
        loss = loss + jnp.mean(pit_res ** 2)

    quantized = pl.pallas_call(
        _final_kernel,
        grid=GRID,
        in_specs=[_row_spec(ART_D), _row_spec(PIT_D), _row_spec(PIT_D)],
        out_specs=_row_spec(D_OUT),
        out_shape=jax.ShapeDtypeStruct((N_TOK, D_OUT), jnp.float32),
        compiler_params=_CP,
    )(art_q, p_qs[0], p_qs[1])

    indices = jnp.concatenate(inds, axis=1).reshape(B, T, ART_Q + PIT_Q)
    return (indices, quantized.reshape(B, T, D_OUT),
            t.reshape(B, T, D_OUT), loss)


# speedup vs baseline: 2.5581x; 2.5581x over previous
import functools

import jax
import jax.numpy as jnp
from jax.experimental import pallas as pl
from jax.experimental.pallas import tpu as pltpu

B, T, D_IN, D_HID, D_OUT = 8, 1024, 512, 512, 256
PITCH_DIM = 8
ART_Q, ART_K, ART_D = 4, 1024, 248
PIT_Q, PIT_K, PIT_D = 2, 256, 8
N_TOK = B * T
BT = 512
GRID = (N_TOK // BT,)


def _dot(a, b):
    return jnp.dot(a, b, preferred_element_type=jnp.float32)


def _rowdot(a, b):
    return jax.lax.dot_general(
        a, b, dimension_numbers=(((1,), (1,)), ((), ())),
        preferred_element_type=jnp.float32)


def _split3(cb):
    mask = jnp.uint32(0xFFFF0000)
    trunc = lambda v: jax.lax.bitcast_convert_type(
        jax.lax.bitcast_convert_type(v, jnp.uint32) & mask, jnp.float32)
    hi = trunc(cb)
    r1 = cb - hi
    mid = trunc(r1)
    lo = r1 - mid
    return hi, mid, lo


def _rowsum(y):
    w = y.shape[1]
    if w >= 128 and w % 128 == 0:
        acc = y[:, 0:128]
        for i in range(1, w // 128):
            acc = acc + y[:, 128 * i:128 * (i + 1)]
    else:
        acc = y
    w = acc.shape[1]
    if w > 8:
        a8 = acc[:, 0:8]
        for c in range(1, w // 8):
            a8 = a8 + acc[:, 8 * c:8 * c + 8]
    else:
        a8 = acc
    a4 = a8[:, 0:4] + a8[:, 4:8]
    a2 = a4[:, 0:2] + a4[:, 2:4]
    return a2[:, 0:1] + a2[:, 1:2]


def _norm_of(v):
    norm = jnp.sqrt(_rowsum(v * v) + 1e-05)
    return jnp.where(norm == 0, 1.0, norm)


def _vq_stage(r, r2, cb_ref, hi_ref, mid_ref, lo_ref, c2_ref, kdim):
    dist = (r2 - 2.0 * _rowdot(r, cb_ref[...])) + c2_ref[...]
    md = jnp.min(dist, axis=1, keepdims=True)
    kiota = jax.lax.broadcasted_iota(jnp.int32, dist.shape, 1)
    ind = jnp.min(jnp.where(dist == md, kiota, kdim), axis=1, keepdims=True)
    onehot = (kiota == ind).astype(jnp.float32)
    q = (_dot(onehot, hi_ref[...]) + _dot(onehot, mid_ref[...])) \
        + _dot(onehot, lo_ref[...])
    r_new = r - q
    return ind, q, r_new, _rowsum(r_new * r_new)


def _fused_kernel(x_ref, W0_ref, b0_ref, Wa1_ref, ba1_ref, Wb1_ref, bb1_ref,
                  W1_ref, b1_ref, Wa2_ref, ba2_ref, Wb2_ref, bb2_ref,
                  Wout_ref, bout_ref,
                  acb_ref, ahi_ref, amid_ref, alo_ref, ac2_ref,
                  pcb_ref, phi_ref, pmid_ref, plo_ref, pc2_ref,
                  ind_ref, quant_ref, t_ref, loss_ref):
    x = x_ref[...]
    ssq = _rowsum(x * x)
    mask = ssq > 0.0
    norm = jnp.sqrt(ssq + 1e-05)
    norm = jnp.where(norm == 0, 1.0, norm)
    x = x / norm

    h = _dot(x, W0_ref[...]) + b0_ref[...]
    h = jnp.maximum(_dot(h, Wa1_ref[...]) + ba1_ref[...], 0.0)
    h = _dot(h, Wb1_ref[...]) + bb1_ref[...]
    h = _dot(h, W1_ref[...]) + b1_ref[...]
    h = jnp.maximum(_dot(h, Wa2_ref[...]) + ba2_ref[...], 0.0)
    h = _dot(h, Wb2_ref[...]) + bb2_ref[...]
    t_pre = _dot(h, Wout_ref[...]) + bout_ref[...]

    ta = t_pre[:, :ART_D]
    tp = t_pre[:, ART_D:]
    ta = ta / _norm_of(ta)
    tp = tp / _norm_of(tp)
    t = jnp.concatenate([ta, tp], axis=1)
    t = jnp.where(mask, t, 0.0)
    t_ref[...] = t

    loss = jnp.zeros((1, 1), jnp.float32)
    inds = []

    r = t[:, :ART_D]
    r2 = _rowsum(r * r)
    art_q = None
    for i in range(ART_Q):
        ind, q, r, r2 = _vq_stage(r, r2, acb_ref.at[i], ahi_ref.at[i],
                                  amid_ref.at[i], alo_ref.at[i],
                                  ac2_ref.at[i], ART_K)
        inds.append(ind)
        art_q = q if art_q is None else art_q + q
        loss = loss + jnp.sum(r2, axis=0, keepdims=True) \
            * (1.0 / (N_TOK * ART_D))

    p = t[:, ART_D:]
    p2 = _rowsum(p * p)
    pit_q = None
    for j in range(PIT_Q):
        ind, q, p, p2 = _vq_stage(p, p2, pcb_ref.at[j], phi_ref.at[j],
                                  pmid_ref.at[j], plo_ref.at[j],
                                  pc2_ref.at[j], PIT_K)
        inds.append(ind)
        pit_q = q if pit_q is None else pit_q + q
        loss = loss + jnp.sum(p2, axis=0, keepdims=True) \
            * (1.0 / (N_TOK * PIT_D))

    ind_ref[...] = jnp.concatenate(inds, axis=1)

    qa = art_q / _norm_of(art_q)
    qp = pit_q / _norm_of(pit_q)
    quant_ref[...] = jnp.concatenate([qa, qp], axis=1)

    @pl.when(pl.program_id(0) == 0)
    def _init():
        loss_ref[...] = jnp.zeros((1, 1), jnp.float32)

    loss_ref[...] = loss_ref[...] + loss


@jax.jit
def kernel(token, W0, b0, Wa1, ba1, Wb1, bb1, W1, b1, Wa2, ba2, Wb2, bb2,
           Wout, bout, art_codebooks, pitch_codebooks):
    x = token.reshape(N_TOK, D_IN)
    ahi, amid, alo = _split3(art_codebooks)
    phi, pmid, plo = _split3(pitch_codebooks)
    ac2 = (art_codebooks ** 2).sum(-1).reshape(ART_Q, 1, ART_K)
    pc2 = (pitch_codebooks ** 2).sum(-1).reshape(PIT_Q, 1, PIT_K)

    row2 = lambda v: v.reshape(1, -1)
    row_spec = lambda w: pl.BlockSpec((BT, w), lambda i: (i, 0))
    w_spec = lambda a: pl.BlockSpec(a.shape, lambda i: (0,) * a.ndim)
    consts = (W0, row2(b0), Wa1, row2(ba1), Wb1, row2(bb1), W1, row2(b1),
              Wa2, row2(ba2), Wb2, row2(bb2), Wout, row2(bout),
              art_codebooks, ahi, amid, alo, ac2,
              pitch_codebooks, phi, pmid, plo, pc2)

    ind_out, quant_out, t_out, loss_out = pl.pallas_call(
        _fused_kernel,
        grid=GRID,
        in_specs=[row_spec(D_IN)] + [w_spec(a) for a in consts],
        out_specs=[
            row_spec(ART_Q + PIT_Q),
            row_spec(D_OUT),
            row_spec(D_OUT),
            pl.BlockSpec((1, 1), lambda i: (0, 0)),
        ],
        out_shape=[
            jax.ShapeDtypeStruct((N_TOK, ART_Q + PIT_Q), jnp.int32),
            jax.ShapeDtypeStruct((N_TOK, D_OUT), jnp.float32),
            jax.ShapeDtypeStruct((N_TOK, D_OUT), jnp.float32),
            jax.ShapeDtypeStruct((1, 1), jnp.float32),
        ],
        compiler_params=pltpu.CompilerParams(
            dimension_semantics=("arbitrary",)),
    )(x, *consts)

    indices = ind_out.reshape(B, T, ART_Q + PIT_Q)
    return (indices, quant_out.reshape(B, T, D_OUT),
            t_out.reshape(B, T, D_OUT), loss_out[0, 0])


# speedup vs baseline: 3.3255x; 1.3000x over previous
import functools

import jax
import jax.numpy as jnp
from jax.experimental import pallas as pl
from jax.experimental.pallas import tpu as pltpu

B, T, D_IN, D_HID, D_OUT = 8, 1024, 512, 512, 256
PITCH_DIM = 8
ART_Q, ART_K, ART_D = 4, 1024, 248
PIT_Q, PIT_K, PIT_D = 2, 256, 8
N_TOK = B * T
BT = 512
GRID = (N_TOK // BT,)


def _dot(a, b):
    return jnp.dot(a, b, preferred_element_type=jnp.float32)


def _rowdot(a, b):
    return jax.lax.dot_general(
        a, b, dimension_numbers=(((1,), (1,)), ((), ())),
        preferred_element_type=jnp.float32)


def _split3(cb):
    mask = jnp.uint32(0xFFFF0000)
    trunc = lambda v: jax.lax.bitcast_convert_type(
        jax.lax.bitcast_convert_type(v, jnp.uint32) & mask, jnp.float32)
    hi = trunc(cb)
    r1 = cb - hi
    mid = trunc(r1)
    lo = r1 - mid
    return hi, mid, lo


def _encoder_kernel(x_ref, W0_ref, b0_ref, Wa1_ref, ba1_ref, Wb1_ref, bb1_ref,
                    W1_ref, b1_ref, Wa2_ref, ba2_ref, Wb2_ref, bb2_ref,
                    Wout_ref, bout_ref, t_ref):
    h = _dot(x_ref[...], W0_ref[...]) + b0_ref[...]
    h = jnp.maximum(_dot(h, Wa1_ref[...]) + ba1_ref[...], 0.0)
    h = _dot(h, Wb1_ref[...]) + bb1_ref[...]
    h = _dot(h, W1_ref[...]) + b1_ref[...]
    h = jnp.maximum(_dot(h, Wa2_ref[...]) + ba2_ref[...], 0.0)
    h = _dot(h, Wb2_ref[...]) + bb2_ref[...]
    t_ref[...] = _dot(h, Wout_ref[...]) + bout_ref[...]


def _vq_stage_kernel(kdim, r_ref, r2_ref, cb_ref, hi_ref, mid_ref, lo_ref,
                     c2_ref, ind_ref, q_ref, rn_ref):
    r = r_ref[...]
    m = _rowdot(r, cb_ref[...])
    dist = (r2_ref[...] - 2.0 * m) + c2_ref[...]
    md = jnp.min(dist, axis=1, keepdims=True)
    kiota = jax.lax.broadcasted_iota(jnp.int32, dist.shape, 1)
    ind = jnp.min(jnp.where(dist == md, kiota, kdim), axis=1, keepdims=True)
    onehot = (kiota == ind).astype(jnp.float32)
    q = (_dot(onehot, hi_ref[...]) + _dot(onehot, mid_ref[...])) \
        + _dot(onehot, lo_ref[...])
    ind_ref[...] = ind
    q_ref[...] = q
    rn_ref[...] = r - q


def _final_kernel(aq_ref, p1_ref, p2_ref, quant_ref):
    pacc = p1_ref[...] + p2_ref[...]
    q = jnp.concatenate([aq_ref[...], pacc], axis=1)
    qa = q[:, :ART_D]
    qp = q[:, ART_D:]
    na = jnp.sqrt((qa ** 2).sum(-1, keepdims=True) + 1e-5)
    na = jnp.where(na == 0.0, 1.0, na)
    npn = jnp.sqrt((qp ** 2).sum(-1, keepdims=True) + 1e-5)
    npn = jnp.where(npn == 0.0, 1.0, npn)
    quant_ref[...] = jnp.concatenate([qa / na, qp / npn], axis=1)


_CP = pltpu.CompilerParams(dimension_semantics=("arbitrary",))
_row_spec = lambda w: pl.BlockSpec((BT, w), lambda i: (i, 0))
_w_spec = lambda a: pl.BlockSpec(a.shape, lambda i: (0,) * a.ndim)


def _vq_stage(r, r2, cb, hi, mid, lo, c2, kdim, ddim):
    return pl.pallas_call(
        functools.partial(_vq_stage_kernel, kdim),
        grid=GRID,
        in_specs=[_row_spec(ddim), _row_spec(1), _w_spec(cb), _w_spec(hi),
                  _w_spec(mid), _w_spec(lo), _w_spec(c2)],
        out_specs=[_row_spec(1), _row_spec(ddim), _row_spec(ddim)],
        out_shape=[
            jax.ShapeDtypeStruct((N_TOK, 1), jnp.int32),
            jax.ShapeDtypeStruct((N_TOK, ddim), jnp.float32),
            jax.ShapeDtypeStruct((N_TOK, ddim), jnp.float32),
        ],
        compiler_params=_CP,
    )(r, r2, cb, hi, mid, lo, c2)


def _unit_norm(x):
    norm = jnp.sqrt((x ** 2).sum(-1, keepdims=True) + 1e-05)
    norm = jnp.where(norm == 0, 1.0, norm)
    return x / norm


@jax.jit
def kernel(token, W0, b0, Wa1, ba1, Wb1, bb1, W1, b1, Wa2, ba2, Wb2, bb2,
           Wout, bout, art_codebooks, pitch_codebooks):
    non_blank_mask = (token ** 2).sum(-1) > 0
    x = _unit_norm(token).reshape(N_TOK, D_IN)

    row2 = lambda v: v.reshape(1, -1)
    enc_args = (W0, row2(b0), Wa1, row2(ba1), Wb1, row2(bb1), W1, row2(b1),
                Wa2, row2(ba2), Wb2, row2(bb2), Wout, row2(bout))
    t_pre = pl.pallas_call(
        _encoder_kernel,
        grid=GRID,
        in_specs=[_row_spec(D_IN)] + [_w_spec(a) for a in enc_args],
        out_specs=_row_spec(D_OUT),
        out_shape=jax.ShapeDtypeStruct((N_TOK, D_OUT), jnp.float32),
        compiler_params=_CP,
    )(x, *enc_args)

    t = jnp.concatenate(
        [_unit_norm(t_pre[..., :-PITCH_DIM]), _unit_norm(t_pre[..., -PITCH_DIM:])], -1)
    t = jnp.where(non_blank_mask.reshape(N_TOK)[..., None], t, 0.0)

    inds = []
    loss = jnp.asarray(0.0, jnp.float32)

    def run_stage(res, cb, kdim, ddim):
        hi, mid, lo = _split3(cb)
        c2 = (cb ** 2).sum(-1).reshape(1, kdim)
        r2 = (res ** 2).sum(-1, keepdims=True)
        return _vq_stage(res, r2, cb, hi, mid, lo, c2, kdim, ddim)

    res = t[:, :ART_D]
    art_q = None
    for i in range(ART_Q):
        ind, q, res = run_stage(res, art_codebooks[i], ART_K, ART_D)
        inds.append(ind)
        art_q = q if art_q is None else art_q + q
        loss = loss + jnp.mean(res ** 2)

    pit_res = t[:, ART_D:]
    p_qs = []
    for j in range(PIT_Q):
        ind, q, pit_res = run_stage(pit_res, pitch_codebooks[j], PIT_K, PIT_D)
        inds.append(ind)
        p_qs.append(q)
        loss = loss + jnp.mean(pit_res ** 2)

    quantized = pl.pallas_call(
        _final_kernel,
        grid=GRID,
        in_specs=[_row_spec(ART_D), _row_spec(PIT_D), _row_spec(PIT_D)],
        out_specs=_row_spec(D_OUT),
        out_shape=jax.ShapeDtypeStruct((N_TOK, D_OUT), jnp.float32),
        compiler_params=_CP,
    )(art_q, p_qs[0], p_qs[1])

    indices = jnp.concatenate(inds, axis=1).reshape(B, T, ART_Q + PIT_Q)
    return (indices, quantized.reshape(B, T, D_OUT),
            t.reshape(B, T, D_OUT), loss)


# speedup vs baseline: 3.5799x; 1.0765x over previous
import functools

import jax
import jax.numpy as jnp
from jax.experimental import pallas as pl
from jax.experimental.pallas import tpu as pltpu

B, T, D_IN, D_HID, D_OUT = 8, 1024, 512, 512, 256
PITCH_DIM = 8
ART_Q, ART_K, ART_D = 4, 1024, 248
PIT_Q, PIT_K, PIT_D = 2, 256, 8
N_TOK = B * T
BT = 1024
GRID = (N_TOK // BT,)


def _dot(a, b):
    return jnp.dot(a, b, preferred_element_type=jnp.float32)


def _rowdot(a, b):
    return jax.lax.dot_general(
        a, b, dimension_numbers=(((1,), (1,)), ((), ())),
        preferred_element_type=jnp.float32)


def _split3(cb):
    mask = jnp.uint32(0xFFFF0000)
    trunc = lambda v: jax.lax.bitcast_convert_type(
        jax.lax.bitcast_convert_type(v, jnp.uint32) & mask, jnp.float32)
    hi = trunc(cb)
    r1 = cb - hi
    mid = trunc(r1)
    lo = r1 - mid
    return hi, mid, lo


def _encoder_kernel(x_ref, W0_ref, b0_ref, Wa1_ref, ba1_ref, Wb1_ref, bb1_ref,
                    W1_ref, b1_ref, Wa2_ref, ba2_ref, Wb2_ref, bb2_ref,
                    Wout_ref, bout_ref, t_ref):
    h = _dot(x_ref[...], W0_ref[...]) + b0_ref[...]
    h = jnp.maximum(_dot(h, Wa1_ref[...]) + ba1_ref[...], 0.0)
    h = _dot(h, Wb1_ref[...]) + bb1_ref[...]
    h = _dot(h, W1_ref[...]) + b1_ref[...]
    h = jnp.maximum(_dot(h, Wa2_ref[...]) + ba2_ref[...], 0.0)
    h = _dot(h, Wb2_ref[...]) + bb2_ref[...]
    t_ref[...] = _dot(h, Wout_ref[...]) + bout_ref[...]


def _vq_stage_kernel(kdim, r_ref, r2_ref, cb_ref, hi_ref, mid_ref, lo_ref,
                     c2_ref, ind_ref, q_ref, rn_ref):
    r = r_ref[...]
    m = _rowdot(r, cb_ref[...])
    dist = (r2_ref[...] - 2.0 * m) + c2_ref[...]
    md = jnp.min(dist, axis=1, keepdims=True)
    kiota = jax.lax.broadcasted_iota(jnp.int32, dist.shape, 1)
    ind = jnp.min(jnp.where(dist == md, kiota, kdim), axis=1, keepdims=True)
    onehot = (kiota == ind).astype(jnp.float32)
    q = (_dot(onehot, hi_ref[...]) + _dot(onehot, mid_ref[...])) \
        + _dot(onehot, lo_ref[...])
    ind_ref[...] = ind
    q_ref[...] = q
    rn_ref[...] = r - q


def _final_kernel(aq_ref, p1_ref, p2_ref, quant_ref):
    pacc = p1_ref[...] + p2_ref[...]
    q = jnp.concatenate([aq_ref[...], pacc], axis=1)
    qa = q[:, :ART_D]
    qp = q[:, ART_D:]
    na = jnp.sqrt((qa ** 2).sum(-1, keepdims=True) + 1e-5)
    na = jnp.where(na == 0.0, 1.0, na)
    npn = jnp.sqrt((qp ** 2).sum(-1, keepdims=True) + 1e-5)
    npn = jnp.where(npn == 0.0, 1.0, npn)
    quant_ref[...] = jnp.concatenate([qa / na, qp / npn], axis=1)


_CP = pltpu.CompilerParams(dimension_semantics=("arbitrary",))
_row_spec = lambda w: pl.BlockSpec((BT, w), lambda i: (i, 0))
_w_spec = lambda a: pl.BlockSpec(a.shape, lambda i: (0,) * a.ndim)


def _vq_stage(r, r2, cb, hi, mid, lo, c2, kdim, ddim):
    return pl.pallas_call(
        functools.partial(_vq_stage_kernel, kdim),
        grid=GRID,
        in_specs=[_row_spec(ddim), _row_spec(1), _w_spec(cb), _w_spec(hi),
                  _w_spec(mid), _w_spec(lo), _w_spec(c2)],
        out_specs=[_row_spec(1), _row_spec(ddim), _row_spec(ddim)],
        out_shape=[
            jax.ShapeDtypeStruct((N_TOK, 1), jnp.int32),
            jax.ShapeDtypeStruct((N_TOK, ddim), jnp.float32),
            jax.ShapeDtypeStruct((N_TOK, ddim), jnp.float32),
        ],
        compiler_params=_CP,
    )(r, r2, cb, hi, mid, lo, c2)


def _unit_norm(x):
    norm = jnp.sqrt((x ** 2).sum(-1, keepdims=True) + 1e-05)
    norm = jnp.where(norm == 0, 1.0, norm)
    return x / norm


@jax.jit
def kernel(token, W0, b0, Wa1, ba1, Wb1, bb1, W1, b1, Wa2, ba2, Wb2, bb2,
           Wout, bout, art_codebooks, pitch_codebooks):
    non_blank_mask = (token ** 2).sum(-1) > 0
    x = _unit_norm(token).reshape(N_TOK, D_IN)

    row2 = lambda v: v.reshape(1, -1)
    enc_args = (W0, row2(b0), Wa1, row2(ba1), Wb1, row2(bb1), W1, row2(b1),
                Wa2, row2(ba2), Wb2, row2(bb2), Wout, row2(bout))
    t_pre = pl.pallas_call(
        _encoder_kernel,
        grid=GRID,
        in_specs=[_row_spec(D_IN)] + [_w_spec(a) for a in enc_args],
        out_specs=_row_spec(D_OUT),
        out_shape=jax.ShapeDtypeStruct((N_TOK, D_OUT), jnp.float32),
        compiler_params=_CP,
    )(x, *enc_args)

    t = jnp.concatenate(
        [_unit_norm(t_pre[..., :-PITCH_DIM]), _unit_norm(t_pre[..., -PITCH_DIM:])], -1)
    t = jnp.where(non_blank_mask.reshape(N_TOK)[..., None], t, 0.0)

    inds = []
    loss = jnp.asarray(0.0, jnp.float32)

    def run_stage(res, cb, kdim, ddim):
        hi, mid, lo = _split3(cb)
        c2 = (cb ** 2).sum(-1).reshape(1, kdim)
        r2 = (res ** 2).sum(-1, keepdims=True)
        return _vq_stage(res, r2, cb, hi, mid, lo, c2, kdim, ddim)

    res = t[:, :ART_D]
    art_q = None
    for i in range(ART_Q):
        ind, q, res = run_stage(res, art_codebooks[i], ART_K, ART_D)
        inds.append(ind)
        art_q = q if art_q is None else art_q + q
        loss = loss + jnp.mean(res ** 2)

    pit_res = t[:, ART_D:]
    p_qs = []
    for j in range(PIT_Q):
        ind, q, pit_res = run_stage(pit_res, pitch_codebooks[j], PIT_K, PIT_D)
        inds.append(ind)
        p_qs.append(q)
        loss = loss + jnp.mean(pit_res ** 2)

    quantized = pl.pallas_call(
        _final_kernel,
        grid=GRID,
        in_specs=[_row_spec(ART_D), _row_spec(PIT_D), _row_spec(PIT_D)],
        out_specs=_row_spec(D_OUT),
        out_shape=jax.ShapeDtypeStruct((N_TOK, D_OUT), jnp.float32),
        compiler_params=_CP,
    )(art_q, p_qs[0], p_qs[1])

    indices = jnp.concatenate(inds, axis=1).reshape(B, T, ART_Q + PIT_Q)
    return (indices, quantized.reshape(B, T, D_OUT),
            t.reshape(B, T, D_OUT), loss)


# speedup vs baseline: 3.5897x; 1.0027x over previous
import functools

import jax
import jax.numpy as jnp
from jax.experimental import pallas as pl
from jax.experimental.pallas import tpu as pltpu

B, T, D_IN, D_HID, D_OUT = 8, 1024, 512, 512, 256
PITCH_DIM = 8
ART_Q, ART_K, ART_D = 4, 1024, 248
PIT_Q, PIT_K, PIT_D = 2, 256, 8
N_TOK = B * T
BT = 2048
GRID = (N_TOK // BT,)


def _dot(a, b):
    return jnp.dot(a, b, preferred_element_type=jnp.float32)


def _rowdot(a, b):
    return jax.lax.dot_general(
        a, b, dimension_numbers=(((1,), (1,)), ((), ())),
        preferred_element_type=jnp.float32)


def _split3(cb):
    mask = jnp.uint32(0xFFFF0000)
    trunc = lambda v: jax.lax.bitcast_convert_type(
        jax.lax.bitcast_convert_type(v, jnp.uint32) & mask, jnp.float32)
    hi = trunc(cb)
    r1 = cb - hi
    mid = trunc(r1)
    lo = r1 - mid
    return hi, mid, lo


def _encoder_kernel(x_ref, W0_ref, b0_ref, Wa1_ref, ba1_ref, Wb1_ref, bb1_ref,
                    W1_ref, b1_ref, Wa2_ref, ba2_ref, Wb2_ref, bb2_ref,
                    Wout_ref, bout_ref, t_ref):
    h = _dot(x_ref[...], W0_ref[...]) + b0_ref[...]
    h = jnp.maximum(_dot(h, Wa1_ref[...]) + ba1_ref[...], 0.0)
    h = _dot(h, Wb1_ref[...]) + bb1_ref[...]
    h = _dot(h, W1_ref[...]) + b1_ref[...]
    h = jnp.maximum(_dot(h, Wa2_ref[...]) + ba2_ref[...], 0.0)
    h = _dot(h, Wb2_ref[...]) + bb2_ref[...]
    t_ref[...] = _dot(h, Wout_ref[...]) + bout_ref[...]


def _vq_stage_kernel(kdim, r_ref, r2_ref, cb_ref, hi_ref, mid_ref, lo_ref,
                     c2_ref, ind_ref, q_ref, rn_ref):
    r = r_ref[...]
    m = _rowdot(r, cb_ref[...])
    dist = (r2_ref[...] - 2.0 * m) + c2_ref[...]
    md = jnp.min(dist, axis=1, keepdims=True)
    kiota = jax.lax.broadcasted_iota(jnp.int32, dist.shape, 1)
    ind = jnp.min(jnp.where(dist == md, kiota, kdim), axis=1, keepdims=True)
    onehot = (kiota == ind).astype(jnp.float32)
    q = (_dot(onehot, hi_ref[...]) + _dot(onehot, mid_ref[...])) \
        + _dot(onehot, lo_ref[...])
    ind_ref[...] = ind
    q_ref[...] = q
    rn_ref[...] = r - q


def _final_kernel(aq_ref, p1_ref, p2_ref, quant_ref):
    pacc = p1_ref[...] + p2_ref[...]
    q = jnp.concatenate([aq_ref[...], pacc], axis=1)
    qa = q[:, :ART_D]
    qp = q[:, ART_D:]
    na = jnp.sqrt((qa ** 2).sum(-1, keepdims=True) + 1e-5)
    na = jnp.where(na == 0.0, 1.0, na)
    npn = jnp.sqrt((qp ** 2).sum(-1, keepdims=True) + 1e-5)
    npn = jnp.where(npn == 0.0, 1.0, npn)
    quant_ref[...] = jnp.concatenate([qa / na, qp / npn], axis=1)


_CP = pltpu.CompilerParams(dimension_semantics=("arbitrary",))
_row_spec = lambda w: pl.BlockSpec((BT, w), lambda i: (i, 0))
_w_spec = lambda a: pl.BlockSpec(a.shape, lambda i: (0,) * a.ndim)


def _vq_stage(r, r2, cb, hi, mid, lo, c2, kdim, ddim):
    return pl.pallas_call(
        functools.partial(_vq_stage_kernel, kdim),
        grid=GRID,
        in_specs=[_row_spec(ddim), _row_spec(1), _w_spec(cb), _w_spec(hi),
                  _w_spec(mid), _w_spec(lo), _w_spec(c2)],
        out_specs=[_row_spec(1), _row_spec(ddim), _row_spec(ddim)],
        out_shape=[
            jax.ShapeDtypeStruct((N_TOK, 1), jnp.int32),
            jax.ShapeDtypeStruct((N_TOK, ddim), jnp.float32),
            jax.ShapeDtypeStruct((N_TOK, ddim), jnp.float32),
        ],
        compiler_params=_CP,
    )(r, r2, cb, hi, mid, lo, c2)


def _unit_norm(x):
    norm = jnp.sqrt((x ** 2).sum(-1, keepdims=True) + 1e-05)
    norm = jnp.where(norm == 0, 1.0, norm)
    return x / norm


@jax.jit
def kernel(token, W0, b0, Wa1, ba1, Wb1, bb1, W1, b1, Wa2, ba2, Wb2, bb2,
           Wout, bout, art_codebooks, pitch_codebooks):
    non_blank_mask = (token ** 2).sum(-1) > 0
    x = _unit_norm(token).reshape(N_TOK, D_IN)

    row2 = lambda v: v.reshape(1, -1)
    enc_args = (W0, row2(b0), Wa1, row2(ba1), Wb1, row2(bb1), W1, row2(b1),
                Wa2, row2(ba2), Wb2, row2(bb2), Wout, row2(bout))
    t_pre = pl.pallas_call(
        _encoder_kernel,
        grid=GRID,
        in_specs=[_row_spec(D_IN)] + [_w_spec(a) for a in enc_args],
        out_specs=_row_spec(D_OUT),
        out_shape=jax.ShapeDtypeStruct((N_TOK, D_OUT), jnp.float32),
        compiler_params=_CP,
    )(x, *enc_args)

    t = jnp.concatenate(
        [_unit_norm(t_pre[..., :-PITCH_DIM]), _unit_norm(t_pre[..., -PITCH_DIM:])], -1)
    t = jnp.where(non_blank_mask.reshape(N_TOK)[..., None], t, 0.0)

    inds = []
    loss = jnp.asarray(0.0, jnp.float32)

    def run_stage(res, cb, kdim, ddim):
        hi, mid, lo = _split3(cb)
        c2 = (cb ** 2).sum(-1).reshape(1, kdim)
        r2 = (res ** 2).sum(-1, keepdims=True)
        return _vq_stage(res, r2, cb, hi, mid, lo, c2, kdim, ddim)

    res = t[:, :ART_D]
    art_q = None
    for i in range(ART_Q):
        ind, q, res = run_stage(res, art_codebooks[i], ART_K, ART_D)
        inds.append(ind)
        art_q = q if art_q is None else art_q + q
        loss = loss + jnp.mean(res ** 2)

    pit_res = t[:, ART_D:]
    p_qs = []
    for j in range(PIT_Q):
        ind, q, pit_res = run_stage(pit_res, pitch_codebooks[j], PIT_K, PIT_D)
        inds.append(ind)
        p_qs.append(q)
        loss = loss + jnp.mean(pit_res ** 2)

    quantized = pl.pallas_call(
        _final_kernel,
        grid=GRID,
        in_specs=[_row_spec(ART_D), _row_spec(PIT_D), _row_spec(PIT_D)],
        out_specs=_row_spec(D_OUT),
        out_shape=jax.ShapeDtypeStruct((N_TOK, D_OUT), jnp.float32),
        compiler_params=_CP,
    )(art_q, p_qs[0], p_qs[1])

    indices = jnp.concatenate(inds, axis=1).reshape(B, T, ART_Q + PIT_Q)
    return (indices, quantized.reshape(B, T, D_OUT),
            t.reshape(B, T, D_OUT), loss)


# speedup vs baseline: 3.8666x; 1.0771x over previous
import functools

import jax
import jax.numpy as jnp
from jax.experimental import pallas as pl
from jax.experimental.pallas import tpu as pltpu

B, T, D_IN, D_HID, D_OUT = 8, 1024, 512, 512, 256
PITCH_DIM = 8
ART_Q, ART_K, ART_D = 4, 1024, 248
PIT_Q, PIT_K, PIT_D = 2, 256, 8
N_TOK = B * T
BT = 2048
GRID = (N_TOK // BT,)


def _dot(a, b):
    return jnp.dot(a, b, preferred_element_type=jnp.float32)


def _rowdot(a, b):
    return jax.lax.dot_general(
        a, b, dimension_numbers=(((1,), (1,)), ((), ())),
        preferred_element_type=jnp.float32)


def _split3(cb):
    mask = jnp.uint32(0xFFFF0000)
    trunc = lambda v: jax.lax.bitcast_convert_type(
        jax.lax.bitcast_convert_type(v, jnp.uint32) & mask, jnp.float32)
    hi = trunc(cb)
    r1 = cb - hi
    mid = trunc(r1)
    lo = r1 - mid
    return hi, mid, lo


def _encoder_kernel(x_ref, W0_ref, b0_ref, Wa1_ref, ba1_ref, Wb1_ref, bb1_ref,
                    W1_ref, b1_ref, Wa2_ref, ba2_ref, Wb2_ref, bb2_ref,
                    Wout_ref, bout_ref, t_ref):
    h = _dot(x_ref[...], W0_ref[...]) + b0_ref[...]
    h = jnp.maximum(_dot(h, Wa1_ref[...]) + ba1_ref[...], 0.0)
    h = _dot(h, Wb1_ref[...]) + bb1_ref[...]
    h = _dot(h, W1_ref[...]) + b1_ref[...]
    h = jnp.maximum(_dot(h, Wa2_ref[...]) + ba2_ref[...], 0.0)
    h = _dot(h, Wb2_ref[...]) + bb2_ref[...]
    t_ref[...] = _dot(h, Wout_ref[...]) + bout_ref[...]


def _vq_stage_kernel(kdim, r_ref, r2_ref, cb_ref, hi_ref, mid_ref, lo_ref,
                     c2_ref, ind_ref, q_ref, rn_ref):
    r = r_ref[...]
    m = _rowdot(r, cb_ref[...])
    dist = (r2_ref[...] - 2.0 * m) + c2_ref[...]
    md = jnp.min(dist, axis=1, keepdims=True)
    kiota = jax.lax.broadcasted_iota(jnp.int32, dist.shape, 1)
    ind = jnp.min(jnp.where(dist == md, kiota, kdim), axis=1, keepdims=True)
    onehot = (kiota == ind).astype(jnp.float32)
    q = (_dot(onehot, hi_ref[...]) + _dot(onehot, mid_ref[...])) \
        + _dot(onehot, lo_ref[...])
    ind_ref[...] = ind
    q_ref[...] = q
    rn_ref[...] = r - q


def _halv8(y):
    a4 = y[:, 0:4] + y[:, 4:8]
    a2 = a4[:, 0:2] + a4[:, 2:4]
    return a2[:, 0:1] + a2[:, 1:2]


def _pitch_substage(p, r2, cb_ref, hi_ref, mid_ref, lo_ref, c2_ref):
    dist = (r2 - 2.0 * _rowdot(p, cb_ref[...])) + c2_ref[...]
    md = jnp.min(dist, axis=1, keepdims=True)
    kiota = jax.lax.broadcasted_iota(jnp.int32, dist.shape, 1)
    ind = jnp.min(jnp.where(dist == md, kiota, PIT_K), axis=1, keepdims=True)
    onehot = (kiota == ind).astype(jnp.float32)
    q = (_dot(onehot, hi_ref[...]) + _dot(onehot, mid_ref[...])) \
        + _dot(onehot, lo_ref[...])
    rn = p - q
    return ind, q, rn, _halv8(rn * rn)


def _pitch_final_kernel(p_ref, aq_ref,
                        cb1_ref, hi1_ref, mid1_ref, lo1_ref, c21_ref,
                        cb2_ref, hi2_ref, mid2_ref, lo2_ref, c22_ref,
                        i1_ref, i2_ref, quant_ref, loss_ref):
    p = p_ref[...]
    r2 = _halv8(p * p)
    i1, q1, rn, r2b = _pitch_substage(p, r2, cb1_ref, hi1_ref, mid1_ref,
                                      lo1_ref, c21_ref)
    i2, q2, rn2, r2c = _pitch_substage(rn, r2b, cb2_ref, hi2_ref, mid2_ref,
                                       lo2_ref, c22_ref)
    i1_ref[...] = i1
    i2_ref[...] = i2

    qa = aq_ref[...]
    qp = q1 + q2
    na = jnp.sqrt((qa ** 2).sum(-1, keepdims=True) + 1e-5)
    na = jnp.where(na == 0.0, 1.0, na)
    npn = jnp.sqrt((qp ** 2).sum(-1, keepdims=True) + 1e-5)
    npn = jnp.where(npn == 0.0, 1.0, npn)
    quant_ref[...] = jnp.concatenate([qa / na, qp / npn], axis=1)

    loss = (jnp.sum(r2b, axis=0, keepdims=True)
            + jnp.sum(r2c, axis=0, keepdims=True)) * (1.0 / (N_TOK * PIT_D))

    @pl.when(pl.program_id(0) == 0)
    def _init():
        loss_ref[...] = jnp.zeros((1, 1), jnp.float32)

    loss_ref[...] = loss_ref[...] + loss


_CP = pltpu.CompilerParams(dimension_semantics=("arbitrary",))
_row_spec = lambda w: pl.BlockSpec((BT, w), lambda i: (i, 0))
_w_spec = lambda a: pl.BlockSpec(a.shape, lambda i: (0,) * a.ndim)


def _vq_stage(r, r2, cb, hi, mid, lo, c2, kdim, ddim):
    return pl.pallas_call(
        functools.partial(_vq_stage_kernel, kdim),
        grid=GRID,
        in_specs=[_row_spec(ddim), _row_spec(1), _w_spec(cb), _w_spec(hi),
                  _w_spec(mid), _w_spec(lo), _w_spec(c2)],
        out_specs=[_row_spec(1), _row_spec(ddim), _row_spec(ddim)],
        out_shape=[
            jax.ShapeDtypeStruct((N_TOK, 1), jnp.int32),
            jax.ShapeDtypeStruct((N_TOK, ddim), jnp.float32),
            jax.ShapeDtypeStruct((N_TOK, ddim), jnp.float32),
        ],
        compiler_params=_CP,
    )(r, r2, cb, hi, mid, lo, c2)


def _unit_norm(x):
    norm = jnp.sqrt((x ** 2).sum(-1, keepdims=True) + 1e-05)
    norm = jnp.where(norm == 0, 1.0, norm)
    return x / norm


@jax.jit
def kernel(token, W0, b0, Wa1, ba1, Wb1, bb1, W1, b1, Wa2, ba2, Wb2, bb2,
           Wout, bout, art_codebooks, pitch_codebooks):
    non_blank_mask = (token ** 2).sum(-1) > 0
    x = _unit_norm(token).reshape(N_TOK, D_IN)

    row2 = lambda v: v.reshape(1, -1)
    enc_args = (W0, row2(b0), Wa1, row2(ba1), Wb1, row2(bb1), W1, row2(b1),
                Wa2, row2(ba2), Wb2, row2(bb2), Wout, row2(bout))
    t_pre = pl.pallas_call(
        _encoder_kernel,
        grid=GRID,
        in_specs=[_row_spec(D_IN)] + [_w_spec(a) for a in enc_args],
        out_specs=_row_spec(D_OUT),
        out_shape=jax.ShapeDtypeStruct((N_TOK, D_OUT), jnp.float32),
        compiler_params=_CP,
    )(x, *enc_args)

    t = jnp.concatenate(
        [_unit_norm(t_pre[..., :-PITCH_DIM]), _unit_norm(t_pre[..., -PITCH_DIM:])], -1)
    t = jnp.where(non_blank_mask.reshape(N_TOK)[..., None], t, 0.0)

    inds = []
    loss = jnp.asarray(0.0, jnp.float32)

    def run_stage(res, cb, kdim, ddim):
        hi, mid, lo = _split3(cb)
        c2 = (cb ** 2).sum(-1).reshape(1, kdim)
        r2 = (res ** 2).sum(-1, keepdims=True)
        return _vq_stage(res, r2, cb, hi, mid, lo, c2, kdim, ddim)

    res = t[:, :ART_D]
    art_q = None
    for i in range(ART_Q):
        ind, q, res = run_stage(res, art_codebooks[i], ART_K, ART_D)
        inds.append(ind)
        art_q = q if art_q is None else art_q + q
        loss = loss + jnp.mean(res ** 2)

    pcb_args = []
    for j in range(PIT_Q):
        cb = pitch_codebooks[j]
        hi, mid, lo = _split3(cb)
        c2 = (cb ** 2).sum(-1).reshape(1, PIT_K)
        pcb_args += [cb, hi, mid, lo, c2]

    i1, i2, quantized, loss_p = pl.pallas_call(
        _pitch_final_kernel,
        grid=GRID,
        in_specs=[_row_spec(PIT_D), _row_spec(ART_D)]
                 + [_w_spec(a) for a in pcb_args],
        out_specs=[_row_spec(1), _row_spec(1), _row_spec(D_OUT),
                   pl.BlockSpec((1, 1), lambda i: (0, 0))],
        out_shape=[
            jax.ShapeDtypeStruct((N_TOK, 1), jnp.int32),
            jax.ShapeDtypeStruct((N_TOK, 1), jnp.int32),
            jax.ShapeDtypeStruct((N_TOK, D_OUT), jnp.float32),
            jax.ShapeDtypeStruct((1, 1), jnp.float32),
        ],
        compiler_params=_CP,
    )(t[:, ART_D:], art_q, *pcb_args)
    inds += [i1, i2]
    loss = loss + loss_p[0, 0]

    indices = jnp.concatenate(inds, axis=1).reshape(B, T, ART_Q + PIT_Q)
    return (indices, quantized.reshape(B, T, D_OUT),
            t.reshape(B, T, D_OUT), loss)
